# baseline (speedup 1.0000x reference)
import functools
import jax
import jax.numpy as jnp
from jax.experimental import pallas as pl
from jax.experimental.pallas import tpu as pltpu
from jax.experimental.pallas import tpu_sc as plsc

L = 16


def _sc_add(x, pos):
    B, S, D = x.shape
    R = 8
    mesh = plsc.VectorSubcoreMesh(core_axis_name="core", subcore_axis_name="subcore")

    @functools.partial(
        pl.kernel,
        out_type=jax.ShapeDtypeStruct((B, S, D), x.dtype),
        mesh=mesh,
        scratch_types=[],
    )
    def k(x_hbm, pos_hbm, out_hbm):
        def body(x_vmem, pos_vmem, out_vmem):
            @pl.loop(0, R)
            def _(r):
                @pl.loop(0, D, step=L)
                def _(c):
                    out_vmem.at[0, r, pl.ds(c, L)][...] = (
                        x_vmem.at[0, r, pl.ds(c, L)][...]
                        + pos_vmem.at[r, pl.ds(c, L)][...]
                    )

        pltpu.emit_pipeline(
            body,
            grid=(B, S // R),
            in_specs=[
                pl.BlockSpec((1, R, D), lambda b, i: (b, i, 0)),
                pl.BlockSpec((R, D), lambda b, i: (i, 0)),
            ],
            out_specs=[pl.BlockSpec((1, R, D), lambda b, i: (b, i, 0))],
            core_axis_name=("core", "subcore"),
            dimension_semantics=(pltpu.PARALLEL, pltpu.PARALLEL),
        )(x_hbm, pos_hbm, out_hbm)

    return k(x, pos)


def kernel(x, pos_table):
    B, S, D = x.shape
    pos = jax.lax.slice(pos_table, (0, 0), (S, D))
    return _sc_add(x, pos)


# speedup vs baseline: 1.6340x; 1.6340x over previous
import functools
import jax
import jax.numpy as jnp
from jax.experimental import pallas as pl
from jax.experimental.pallas import tpu as pltpu
from jax.experimental.pallas import tpu_sc as plsc

L = 16


def _tc_add_kernel(x_ref, pos_ref, out_ref):
    out_ref[...] = x_ref[...] + pos_ref[...][None, :, :]


def _tc_part(x, pos, nb):
    B, S, D = x.shape
    BS = 512
    return pl.pallas_call(
        _tc_add_kernel,
        grid=(S // BS,),
        in_specs=[
            pl.BlockSpec((nb, BS, D), lambda i: (0, i, 0)),
            pl.BlockSpec((BS, D), lambda i: (i, 0)),
        ],
        out_specs=pl.BlockSpec((nb, BS, D), lambda i: (0, i, 0)),
        out_shape=jax.ShapeDtypeStruct((nb, S, D), x.dtype),
    )(x, pos)


def _sc_part(x, pos, b0):
    B, S, D = x.shape
    R = 8
    mesh = plsc.VectorSubcoreMesh(core_axis_name="core", subcore_axis_name="subcore")

    @functools.partial(
        pl.kernel,
        out_type=jax.ShapeDtypeStruct((B - b0, S, D), x.dtype),
        mesh=mesh,
        scratch_types=[],
    )
    def k(x_hbm, pos_hbm, out_hbm):
        def body(x_vmem, pos_vmem, out_vmem):
            @pl.loop(0, R)
            def _(r):
                @pl.loop(0, D, step=L)
                def _(c):
                    out_vmem.at[0, r, pl.ds(c, L)][...] = (
                        x_vmem.at[0, r, pl.ds(c, L)][...]
                        + pos_vmem.at[r, pl.ds(c, L)][...]
                    )

        pltpu.emit_pipeline(
            body,
            grid=(B - b0, S // R),
            in_specs=[
                pl.BlockSpec((1, R, D), lambda b, i: (b0 + b, i, 0)),
                pl.BlockSpec((R, D), lambda b, i: (i, 0)),
            ],
            out_specs=[pl.BlockSpec((1, R, D), lambda b, i: (b, i, 0))],
            core_axis_name=("core", "subcore"),
            dimension_semantics=(pltpu.PARALLEL, pltpu.PARALLEL),
        )(x_hbm, pos_hbm, out_hbm)

    return k(x, pos)


def kernel(x, pos_table):
    B, S, D = x.shape
    pos = jax.lax.slice(pos_table, (0, 0), (S, D))
    nb_tc = 3
    tc_out = _tc_part(x, pos, nb_tc)
    sc_out = _sc_part(x, pos, nb_tc)
    return jnp.concatenate([tc_out, sc_out], axis=0)


# speedup vs baseline: 2.4616x; 1.5064x over previous
import functools
import jax
import jax.numpy as jnp
from jax.experimental import pallas as pl
from jax.experimental.pallas import tpu as pltpu
from jax.experimental.pallas import tpu_sc as plsc

L = 16


def _sc_add(x, pos):
    B, S, D = x.shape
    R = 4
    mesh = plsc.VectorSubcoreMesh(core_axis_name="core", subcore_axis_name="subcore")

    @functools.partial(
        pl.kernel,
        out_type=jax.ShapeDtypeStruct((B, S, D), x.dtype),
        mesh=mesh,
        scratch_types=[],
    )
    def k(x_hbm, pos_hbm, out_hbm):
        def body(x_vmem, pos_vmem, out_vmem):
            for r in range(R):
                @plsc.parallel_loop(0, D, L, unroll=8)
                def _(c):
                    pv = pos_vmem.at[r, pl.ds(c, L)][...]
                    for b in range(B):
                        out_vmem.at[b, r, pl.ds(c, L)][...] = (
                            x_vmem.at[b, r, pl.ds(c, L)][...] + pv
                        )

        pltpu.emit_pipeline(
            body,
            grid=(S // R,),
            in_specs=[
                pl.BlockSpec((B, R, D), lambda i: (0, i, 0)),
                pl.BlockSpec((R, D), lambda i: (i, 0)),
            ],
            out_specs=[pl.BlockSpec((B, R, D), lambda i: (0, i, 0))],
            core_axis_name=("core", "subcore"),
            dimension_semantics=(pltpu.PARALLEL,),
        )(x_hbm, pos_hbm, out_hbm)

    return k(x, pos)


def kernel(x, pos_table):
    B, S, D = x.shape
    pos = jax.lax.slice(pos_table, (0, 0), (S, D))
    return _sc_add(x, pos)


# speedup vs baseline: 3.3440x; 1.3585x over previous
import jax
import jax.numpy as jnp
from jax.experimental import pallas as pl


def _add_kernel(x0_ref, x1_ref, x2_ref, x3_ref, pos_ref, out_ref):
    p = pos_ref[...]
    out_ref[0] = x0_ref[0] + p
    out_ref[1] = x1_ref[0] + p
    out_ref[2] = x2_ref[0] + p
    out_ref[3] = x3_ref[0] + p


def kernel(x, pos_table):
    B, S, D = x.shape
    pos = jax.lax.slice(pos_table, (0, 0), (S, D))
    BS = 512
    xspec = lambda b: pl.BlockSpec((1, BS, D), lambda i: (b, i, 0))
    return pl.pallas_call(
        _add_kernel,
        grid=(S // BS,),
        in_specs=[xspec(0), xspec(1), xspec(2), xspec(3),
                  pl.BlockSpec((BS, D), lambda i: (i, 0))],
        out_specs=pl.BlockSpec((B, BS, D), lambda i: (0, i, 0)),
        out_shape=jax.ShapeDtypeStruct((B, S, D), x.dtype),
    )(x, x, x, x, pos)


# speedup vs baseline: 4.2378x; 1.2673x over previous
import jax
import jax.numpy as jnp
from jax.experimental import pallas as pl


def _add_kernel(x_ref, pos_ref, out_ref):
    out_ref[...] = x_ref[...] + pos_ref[...][None, :, :]


def kernel(x, pos_table):
    B, S, D = x.shape
    BS = 512
    return pl.pallas_call(
        _add_kernel,
        grid=(S // BS,),
        in_specs=[
            pl.BlockSpec((B, BS, D), lambda i: (0, i, 0)),
            pl.BlockSpec((BS, D), lambda i: (i, 0)),
        ],
        out_specs=pl.BlockSpec((B, BS, D), lambda i: (0, i, 0)),
        out_shape=jax.ShapeDtypeStruct((B, S, D), x.dtype),
    )(x, pos_table)
